# baseline (speedup 1.0000x reference)
import functools

import numpy as np
import jax
import jax.numpy as jnp
from jax import lax
from jax.experimental import pallas as pl
from jax.experimental.pallas import tpu as pltpu
from jax.experimental.pallas import tpu_sc as plsc

N = 100000
E = 1600000
NF = 64
DW = 8

NC = 2
NS = 16
NW = NC * NS

G_KB = 16
G_CHUNKS = 49
G_TILE_STREAMS = G_KB * G_CHUNKS
G_STREAMS = NW * G_TILE_STREAMS
G_PAD = G_STREAMS * 128

E_BLK = 6400
S_KB = 16
S_CHUNKS = 25
S_TILE_STREAMS = S_KB * S_CHUNKS
S_STREAMS = NW * S_TILE_STREAMS
E_PAD = S_STREAMS * 128
MLP_GRID = E_PAD // E_BLK
MLP_VALID = E // E_BLK

NSL = 6248
N_EXTRA = N - NS * NSL

N_PAD = 100352
NL = N_PAD // 16
RL = E_BLK // 16
EL = E_PAD // 16
GL = G_PAD // 16
UB = NL // 4


def _sc_mesh():
    return plsc.VectorSubcoreMesh(core_axis_name="c", subcore_axis_name="s",
                                  num_cores=NC, num_subcores=NS)


_SC_PARAMS = pltpu.CompilerParams(use_tc_tiling_on_sc=False)


def _gather_body(x_hbm, idx_hbm, out_hbm, idx_v, rows_v, sem):
    c = lax.axis_index("c")
    s = lax.axis_index("s")
    w = s * NC + c
    row0 = w * G_TILE_STREAMS

    def chunk(k, _):
        r0 = row0 + k * G_KB
        pltpu.sync_copy(idx_hbm.at[pl.ds(r0 * 128, G_KB * 128)], idx_v)
        descs = []
        for j in range(G_KB):
            descs.append(pltpu.async_copy(
                x_hbm.at[idx_v.at[pl.ds(j * 128, 128)]],
                rows_v.at[pl.ds(j * 128, 128)], sem))
        for d in descs:
            d.wait()
        pltpu.sync_copy(rows_v, out_hbm.at[pl.ds(r0 * 128, G_KB * 128)])
        return 0

    lax.fori_loop(0, G_CHUNKS, chunk, 0)


def _gather(x4, idx2d):
    return pl.kernel(
        _gather_body,
        out_type=jax.ShapeDtypeStruct((G_PAD, DW), jnp.float32),
        mesh=_sc_mesh(),
        compiler_params=_SC_PARAMS,
        scratch_types=[
            pltpu.VMEM((G_KB * 128,), jnp.int32),
            pltpu.VMEM((G_KB * 128, DW), jnp.float32),
            pltpu.SemaphoreType.DMA,
        ],
    )(x4, idx2d)


def _scatter_body(m_hbm, idx_hbm, zero4_hbm, out_hbm, idx_v, m_v, acc):
    c = lax.axis_index("c")
    s = lax.axis_index("s")
    w = s * NC + c
    z0 = s * NSL
    pltpu.sync_copy(zero4_hbm.at[pl.ds(z0, NSL)], acc.at[pl.ds(z0, NSL)])

    @pl.when(s == NS - 1)
    def _():
        pltpu.sync_copy(zero4_hbm.at[pl.ds(NS * NSL, N_EXTRA)],
                        acc.at[pl.ds(NS * NSL, N_EXTRA)])

    plsc.subcore_barrier()
    row0 = w * S_TILE_STREAMS

    def chunk(k, _):
        r0 = row0 + k * S_KB
        pltpu.sync_copy(idx_hbm.at[pl.ds(r0 * 128, S_KB * 128)], idx_v)
        pltpu.sync_copy(m_hbm.at[pl.ds(r0 * 128, S_KB * 128)], m_v)
        for j in range(S_KB):
            pltpu.sync_copy(m_v.at[pl.ds(j * 128, 128)],
                            acc.at[idx_v.at[pl.ds(j * 128, 128)]], add=True)
        return 0

    lax.fori_loop(0, S_CHUNKS, chunk, 0)
    plsc.subcore_barrier()
    pltpu.sync_copy(acc.at[pl.ds(z0, NSL)], out_hbm.at[c, pl.ds(z0, NSL)])

    @pl.when(s == NS - 1)
    def _():
        pltpu.sync_copy(acc.at[pl.ds(NS * NSL, N_EXTRA)],
                        out_hbm.at[c, pl.ds(NS * NSL, N_EXTRA)])


def _scatter(m, idx2d, zeros4):
    return pl.kernel(
        _scatter_body,
        out_type=jax.ShapeDtypeStruct((NC, N_PAD, DW), jnp.float32),
        mesh=_sc_mesh(),
        compiler_params=_SC_PARAMS,
        scratch_types=[
            pltpu.VMEM((S_KB * 128,), jnp.int32),
            pltpu.VMEM((S_KB * 128, DW), jnp.float32),
            pltpu.VMEM_SHARED((N, DW), jnp.float32),
        ],
    )(m, idx2d, zeros4)


def _count_body(idx_hbm, zero1_hbm, out0_hbm, out1_hbm, idx_v, ones_v, acc):
    c = lax.axis_index("c")
    s = lax.axis_index("s")
    w = s * NC + c
    for t in range(8):
        ones_v[pl.ds(t * 16, 16)] = jnp.full((16,), 1.0, jnp.float32)
    z0 = s * NSL
    pltpu.sync_copy(zero1_hbm.at[pl.ds(z0, NSL)], acc.at[pl.ds(z0, NSL)])

    @pl.when(s == NS - 1)
    def _():
        pltpu.sync_copy(zero1_hbm.at[pl.ds(NS * NSL, N_EXTRA)],
                        acc.at[pl.ds(NS * NSL, N_EXTRA)])

    plsc.subcore_barrier()
    row0 = w * S_TILE_STREAMS

    def chunk(k, _):
        r0 = row0 + k * S_KB
        pltpu.sync_copy(idx_hbm.at[pl.ds(r0 * 128, S_KB * 128)], idx_v)
        for j in range(S_KB):
            pltpu.sync_copy(ones_v, acc.at[idx_v.at[pl.ds(j * 128, 128)]],
                            add=True)
        return 0

    lax.fori_loop(0, S_CHUNKS, chunk, 0)
    plsc.subcore_barrier()

    @pl.when(c == 0)
    def _():
        pltpu.sync_copy(acc.at[pl.ds(z0, NSL)], out0_hbm.at[pl.ds(z0, NSL)])

        @pl.when(s == NS - 1)
        def _():
            pltpu.sync_copy(acc.at[pl.ds(NS * NSL, N_EXTRA)],
                            out0_hbm.at[pl.ds(NS * NSL, N_EXTRA)])

    @pl.when(c == 1)
    def _():
        pltpu.sync_copy(acc.at[pl.ds(z0, NSL)], out1_hbm.at[pl.ds(z0, NSL)])

        @pl.when(s == NS - 1)
        def _():
            pltpu.sync_copy(acc.at[pl.ds(NS * NSL, N_EXTRA)],
                            out1_hbm.at[pl.ds(NS * NSL, N_EXTRA)])


def _count(idx2d, zeros1):
    return pl.kernel(
        _count_body,
        out_type=(jax.ShapeDtypeStruct((N,), jnp.float32),
                  jax.ShapeDtypeStruct((N,), jnp.float32)),
        mesh=_sc_mesh(),
        compiler_params=_SC_PARAMS,
        scratch_types=[
            pltpu.VMEM((S_KB * 128,), jnp.int32),
            pltpu.VMEM((128,), jnp.float32),
            pltpu.VMEM_SHARED((N,), jnp.float32),
        ],
    )(idx2d, zeros1)


def _edge_mlp_body(xr_ref, xc_ref, ea_ref, w_ref, b_ref, v_ref, s_ref,
                   r_ref, bb_ref, out_ref):
    pid = pl.program_id(0)

    @pl.when(pid < MLP_VALID)
    def _():
        xd = xr_ref[...] - xc_ref[...]
        r2 = jnp.dot(xd * xd, s_ref[...],
                     preferred_element_type=jnp.float32)
        radial = jnp.sqrt(r2)
        f = jnp.dot(radial, r_ref[...],
                    preferred_element_type=jnp.float32) + ea_ref[...]
        h = jnp.dot(f, w_ref[...],
                    preferred_element_type=jnp.float32) + b_ref[...]
        hs = h * (1.0 / (1.0 + jnp.exp(-h)))
        z = jnp.dot(hs, v_ref[...], preferred_element_type=jnp.float32)
        e = jnp.tanh(z)
        out_ref[...] = xd * jnp.dot(e, bb_ref[...],
                                    preferred_element_type=jnp.float32)

    @pl.when(pid >= MLP_VALID)
    def _():
        out_ref[...] = jnp.zeros_like(out_ref)


def _edge_mlp(g128, ea_pack, w_big, b_big, v_big, s_mat, r_mat, bb_mat):
    cl = lambda i: jnp.minimum(i, MLP_VALID - 1)
    return pl.pallas_call(
        _edge_mlp_body,
        grid=(MLP_GRID,),
        in_specs=[
            pl.BlockSpec((RL, 128), lambda i: (cl(i), 0)),
            pl.BlockSpec((RL, 128), lambda i: (cl(i) + MLP_VALID, 0)),
            pl.BlockSpec((RL, 128), lambda i: (cl(i), 0)),
            pl.BlockSpec((128, 1024), lambda i: (0, 0)),
            pl.BlockSpec((1, 1024), lambda i: (0, 0)),
            pl.BlockSpec((1024, 16), lambda i: (0, 0)),
            pl.BlockSpec((128, 16), lambda i: (0, 0)),
            pl.BlockSpec((16, 128), lambda i: (0, 0)),
            pl.BlockSpec((16, 128), lambda i: (0, 0)),
        ],
        out_specs=pl.BlockSpec((RL, 128), lambda i: (i, 0)),
        out_shape=jax.ShapeDtypeStruct((EL, 128), jnp.float32),
    )(g128, g128, ea_pack, w_big, b_big, v_big, s_mat, r_mat, bb_mat)


def _update_body(x_ref, p_ref, ic_ref, vel_ref, vn_ref, wv_ref, bv_ref,
                 vv_ref, c2b_ref, bb_ref, out_ref):
    agg = (p_ref[0] + p_ref[1]) * ic_ref[...]
    h = jnp.dot(vn_ref[...], wv_ref[...],
                preferred_element_type=jnp.float32) + bv_ref[...]
    hs = h * (1.0 / (1.0 + jnp.exp(-h)))
    z = jnp.dot(hs, vv_ref[...],
                preferred_element_type=jnp.float32) + c2b_ref[...]
    scale = jnp.dot(z, bb_ref[...], preferred_element_type=jnp.float32)
    out_ref[...] = x_ref[...] + agg + vel_ref[...] * scale


def _update(x128, p128, ic, vel128, vn, wv, bv, vv, c2b, bb_mat):
    return pl.pallas_call(
        _update_body,
        grid=(NL // UB,),
        in_specs=[
            pl.BlockSpec((UB, 128), lambda i: (i, 0)),
            pl.BlockSpec((NC, UB, 128), lambda i: (0, i, 0)),
            pl.BlockSpec((UB, 128), lambda i: (i, 0)),
            pl.BlockSpec((UB, 128), lambda i: (i, 0)),
            pl.BlockSpec((UB, 128), lambda i: (i, 0)),
            pl.BlockSpec((128, 1024), lambda i: (0, 0)),
            pl.BlockSpec((1, 1024), lambda i: (0, 0)),
            pl.BlockSpec((1024, 16), lambda i: (0, 0)),
            pl.BlockSpec((1, 1), lambda i: (0, 0)),
            pl.BlockSpec((16, 128), lambda i: (0, 0)),
        ],
        out_specs=pl.BlockSpec((UB, 128), lambda i: (i, 0)),
        out_shape=jax.ShapeDtypeStruct((NL, 128), jnp.float32),
    )(x128, p128, ic, vel128, vn, wv, bv, vv, c2b, bb_mat)


_I16 = np.eye(16, dtype=np.float32)


def _big_w(w1):
    blocks = []
    for t in range(8):
        p = jnp.pad(w1[8 * t:8 * t + 8, :].T, ((0, 3), (0, 0)))
        blocks.append(jnp.kron(_I16, p))
    return jnp.concatenate(blocks, axis=1)


def _big_b(b1):
    return jnp.concatenate(
        [jnp.tile(b1[8 * t:8 * t + 8], 16) for t in range(8)])[None, :]


def _big_v(v2):
    return jnp.concatenate(
        [jnp.kron(_I16, v2[8 * t:8 * t + 8][:, None]) for t in range(8)],
        axis=0)


def kernel(vel_norm, x, edges, vel, edge_attr, phi1_w, phi1_b, phi2_w,
           cmv1_w, cmv1_b, cmv2_w, cmv2_b):
    row = edges[0]
    col = edges[1]
    idx_g = jnp.concatenate(
        [row, col, jnp.zeros((G_PAD - 2 * E,), jnp.int32)])
    idx_s = jnp.concatenate([row, jnp.zeros((E_PAD - E,), jnp.int32)])
    zeros4 = jnp.zeros((N, DW), jnp.float32)
    zeros1 = jnp.zeros((N,), jnp.float32)

    s_mat = jnp.asarray(np.kron(_I16, np.ones((8, 1), np.float32)))
    r_mat = jnp.asarray(np.kron(_I16, np.eye(8, dtype=np.float32)[:1]))
    bb_mat = jnp.asarray(np.kron(_I16, np.ones((1, 8), np.float32)))

    x8 = jnp.pad(x, ((0, N_PAD - N), (0, DW - 3)))
    vel128 = jnp.pad(vel, ((0, N_PAD - N), (0, DW - 3))).reshape(NL, 128)
    vn128 = jnp.pad(vel_norm, ((0, N_PAD - N), (0, DW - 1))).reshape(NL, 128)
    ea_pack = jnp.pad(edge_attr, ((0, 0), (1, 3))).reshape(E // 16, 128)

    cnt0, cnt1 = _count(idx_s, zeros1)
    cnt = cnt0 + cnt1
    cnt = cnt.at[0].add(-float(E_PAD - E))
    inv_cnt = jnp.pad(1.0 / jnp.clip(cnt, 1.0, None), (0, N_PAD - N))
    ic128 = jnp.repeat(inv_cnt.reshape(NL, 16), 8, axis=1)

    for i in range(4):
        gathered = _gather(x8, idx_g)
        g128 = gathered.reshape(GL, 128)
        w_big = _big_w(phi1_w[i])
        b_big = _big_b(phi1_b[i])
        v_big = _big_v(phi2_w[i][0])
        m128 = _edge_mlp(g128, ea_pack, w_big, b_big, v_big,
                         s_mat, r_mat, bb_mat)
        p = _scatter(m128.reshape(E_PAD, DW), idx_s, zeros4)
        wv = _big_w(jnp.pad(cmv1_w[i], ((0, 0), (0, 4))))
        bv = _big_b(cmv1_b[i])
        vv = _big_v(cmv2_w[i][0])
        x128 = _update(x8.reshape(NL, 128), p.reshape(NC, NL, 128),
                       ic128, vel128, vn128, wv, bv, vv,
                       cmv2_b[i][:, None], bb_mat)
        x8 = x128.reshape(N_PAD, DW)
    return x8[:N, :3]


# speedup vs baseline: 1.7142x; 1.7142x over previous
import functools

import numpy as np
import jax
import jax.numpy as jnp
from jax import lax
from jax.experimental import pallas as pl
from jax.experimental.pallas import tpu as pltpu
from jax.experimental.pallas import tpu_sc as plsc

N = 100000
E = 1600000
NF = 64
DW = 8

NC = 2
NS = 16
NW = NC * NS

G_KB = 16
G_CHUNKS = 49
G_TILE_STREAMS = G_KB * G_CHUNKS
G_STREAMS = NW * G_TILE_STREAMS
G_PAD = G_STREAMS * 128

E_BLK = 6400
S_KB = 16
S_CHUNKS = 25
S_TILE_STREAMS = S_KB * S_CHUNKS
S_STREAMS = NW * S_TILE_STREAMS
E_PAD = S_STREAMS * 128
MLP_GRID = E_PAD // E_BLK
MLP_VALID = E // E_BLK

NSL = 6248
N_EXTRA = N - NS * NSL

N_PAD = 100352
NL = N_PAD // 16
RL = E_BLK // 16
EL = E_PAD // 16
GL = G_PAD // 16
UB = NL // 4


def _sc_mesh():
    return plsc.VectorSubcoreMesh(core_axis_name="c", subcore_axis_name="s",
                                  num_cores=NC, num_subcores=NS)


_SC_PARAMS = pltpu.CompilerParams(use_tc_tiling_on_sc=False)


def _gather_body(x_hbm, idx_hbm, out_hbm, idx_v, rows_v, sem):
    c = lax.axis_index("c")
    s = lax.axis_index("s")
    w = s * NC + c
    row0 = w * G_TILE_STREAMS

    def chunk(k, _):
        r0 = row0 + k * G_KB
        pltpu.sync_copy(idx_hbm.at[pl.ds(r0 * 128, G_KB * 128)], idx_v)
        descs = []
        for j in range(G_KB):
            descs.append(pltpu.async_copy(
                x_hbm.at[idx_v.at[pl.ds(j * 128, 128)]],
                rows_v.at[pl.ds(j * 128, 128)], sem))
        for d in descs:
            d.wait()
        pltpu.sync_copy(rows_v, out_hbm.at[pl.ds(r0 * 128, G_KB * 128)])
        return 0

    lax.fori_loop(0, G_CHUNKS, chunk, 0)


def _gather(x4, idx2d):
    return pl.kernel(
        _gather_body,
        out_type=jax.ShapeDtypeStruct((G_PAD, DW), jnp.float32),
        mesh=_sc_mesh(),
        compiler_params=_SC_PARAMS,
        scratch_types=[
            pltpu.VMEM((G_KB * 128,), jnp.int32),
            pltpu.VMEM((G_KB * 128, DW), jnp.float32),
            pltpu.SemaphoreType.DMA,
        ],
    )(x4, idx2d)


def _scatter_body(m_hbm, idx_hbm, zero4_hbm, out_hbm, idx_v, m_v, acc):
    c = lax.axis_index("c")
    s = lax.axis_index("s")
    w = s * NC + c
    z0 = s * NSL
    pltpu.sync_copy(zero4_hbm.at[pl.ds(z0, NSL)], acc.at[pl.ds(z0, NSL)])

    @pl.when(s == NS - 1)
    def _():
        pltpu.sync_copy(zero4_hbm.at[pl.ds(NS * NSL, N_EXTRA)],
                        acc.at[pl.ds(NS * NSL, N_EXTRA)])

    plsc.subcore_barrier()
    row0 = w * S_TILE_STREAMS

    def chunk(k, _):
        r0 = row0 + k * S_KB
        pltpu.sync_copy(idx_hbm.at[pl.ds(r0 * 128, S_KB * 128)], idx_v)
        pltpu.sync_copy(m_hbm.at[pl.ds(r0 * 128, S_KB * 128)], m_v)
        for j in range(S_KB):
            pltpu.sync_copy(m_v.at[pl.ds(j * 128, 128)],
                            acc.at[idx_v.at[pl.ds(j * 128, 128)]], add=True)
        return 0

    lax.fori_loop(0, S_CHUNKS, chunk, 0)
    plsc.subcore_barrier()
    pltpu.sync_copy(acc.at[pl.ds(z0, NSL)], out_hbm.at[c, pl.ds(z0, NSL)])

    @pl.when(s == NS - 1)
    def _():
        pltpu.sync_copy(acc.at[pl.ds(NS * NSL, N_EXTRA)],
                        out_hbm.at[c, pl.ds(NS * NSL, N_EXTRA)])


def _scatter(m, idx2d, zeros4):
    return pl.kernel(
        _scatter_body,
        out_type=jax.ShapeDtypeStruct((NC, N_PAD, DW), jnp.float32),
        mesh=_sc_mesh(),
        compiler_params=_SC_PARAMS,
        scratch_types=[
            pltpu.VMEM((S_KB * 128,), jnp.int32),
            pltpu.VMEM((S_KB * 128, DW), jnp.float32),
            pltpu.VMEM_SHARED((N, DW), jnp.float32),
        ],
    )(m, idx2d, zeros4)


def _count_body(idx_hbm, zero1_hbm, out0_hbm, out1_hbm, idx_v, ones_v, acc):
    c = lax.axis_index("c")
    s = lax.axis_index("s")
    w = s * NC + c
    for t in range(8):
        ones_v[pl.ds(t * 16, 16)] = jnp.full((16,), 1.0, jnp.float32)
    z0 = s * NSL
    pltpu.sync_copy(zero1_hbm.at[pl.ds(z0, NSL)], acc.at[pl.ds(z0, NSL)])

    @pl.when(s == NS - 1)
    def _():
        pltpu.sync_copy(zero1_hbm.at[pl.ds(NS * NSL, N_EXTRA)],
                        acc.at[pl.ds(NS * NSL, N_EXTRA)])

    plsc.subcore_barrier()
    row0 = w * S_TILE_STREAMS

    def chunk(k, _):
        r0 = row0 + k * S_KB
        pltpu.sync_copy(idx_hbm.at[pl.ds(r0 * 128, S_KB * 128)], idx_v)
        for j in range(S_KB):
            pltpu.sync_copy(ones_v, acc.at[idx_v.at[pl.ds(j * 128, 128)]],
                            add=True)
        return 0

    lax.fori_loop(0, S_CHUNKS, chunk, 0)
    plsc.subcore_barrier()

    @pl.when(c == 0)
    def _():
        pltpu.sync_copy(acc.at[pl.ds(z0, NSL)], out0_hbm.at[pl.ds(z0, NSL)])

        @pl.when(s == NS - 1)
        def _():
            pltpu.sync_copy(acc.at[pl.ds(NS * NSL, N_EXTRA)],
                            out0_hbm.at[pl.ds(NS * NSL, N_EXTRA)])

    @pl.when(c == 1)
    def _():
        pltpu.sync_copy(acc.at[pl.ds(z0, NSL)], out1_hbm.at[pl.ds(z0, NSL)])

        @pl.when(s == NS - 1)
        def _():
            pltpu.sync_copy(acc.at[pl.ds(NS * NSL, N_EXTRA)],
                            out1_hbm.at[pl.ds(NS * NSL, N_EXTRA)])


def _count(idx2d, zeros1):
    return pl.kernel(
        _count_body,
        out_type=(jax.ShapeDtypeStruct((N,), jnp.float32),
                  jax.ShapeDtypeStruct((N,), jnp.float32)),
        mesh=_sc_mesh(),
        compiler_params=_SC_PARAMS,
        scratch_types=[
            pltpu.VMEM((S_KB * 128,), jnp.int32),
            pltpu.VMEM((128,), jnp.float32),
            pltpu.VMEM_SHARED((N,), jnp.float32),
        ],
    )(idx2d, zeros1)


def _edge_mlp_body(xr_ref, xc_ref, ea_ref, w_ref, b_ref, v_ref, s_ref,
                   r_ref, bb_ref, out_ref):
    pid = pl.program_id(0)

    @pl.when(pid < MLP_VALID)
    def _():
        xd = xr_ref[...] - xc_ref[...]
        r2 = jnp.dot(xd * xd, s_ref[...],
                     preferred_element_type=jnp.float32)
        radial = jnp.sqrt(r2)
        f = jnp.dot(radial, r_ref[...],
                    preferred_element_type=jnp.float32) + ea_ref[...]
        h = jnp.dot(f, w_ref[...],
                    preferred_element_type=jnp.float32) + b_ref[...]
        hs = h * (1.0 / (1.0 + jnp.exp(-h)))
        z = jnp.dot(hs, v_ref[...], preferred_element_type=jnp.float32)
        e = jnp.tanh(z)
        out_ref[...] = xd * jnp.dot(e, bb_ref[...],
                                    preferred_element_type=jnp.float32)

    @pl.when(pid >= MLP_VALID)
    def _():
        out_ref[...] = jnp.zeros_like(out_ref)


def _edge_mlp(g128, ea_pack, w_big, b_big, v_big, s_mat, r_mat, bb_mat):
    cl = lambda i: jnp.minimum(i, MLP_VALID - 1)
    return pl.pallas_call(
        _edge_mlp_body,
        grid=(MLP_GRID,),
        in_specs=[
            pl.BlockSpec((RL, 128), lambda i: (cl(i), 0)),
            pl.BlockSpec((RL, 128), lambda i: (cl(i) + MLP_VALID, 0)),
            pl.BlockSpec((RL, 128), lambda i: (cl(i), 0)),
            pl.BlockSpec((128, 1024), lambda i: (0, 0)),
            pl.BlockSpec((1, 1024), lambda i: (0, 0)),
            pl.BlockSpec((1024, 16), lambda i: (0, 0)),
            pl.BlockSpec((128, 16), lambda i: (0, 0)),
            pl.BlockSpec((16, 128), lambda i: (0, 0)),
            pl.BlockSpec((16, 128), lambda i: (0, 0)),
        ],
        out_specs=pl.BlockSpec((RL, 128), lambda i: (i, 0)),
        out_shape=jax.ShapeDtypeStruct((EL, 128), jnp.float32),
    )(g128, g128, ea_pack, w_big, b_big, v_big, s_mat, r_mat, bb_mat)


def _update_body(x_ref, p_ref, ic_ref, vel_ref, vn_ref, wv_ref, bv_ref,
                 vv_ref, c2b_ref, bb_ref, out_ref):
    agg = (p_ref[0] + p_ref[1]) * ic_ref[...]
    h = jnp.dot(vn_ref[...], wv_ref[...],
                preferred_element_type=jnp.float32) + bv_ref[...]
    hs = h * (1.0 / (1.0 + jnp.exp(-h)))
    z = jnp.dot(hs, vv_ref[...],
                preferred_element_type=jnp.float32) + c2b_ref[...]
    scale = jnp.dot(z, bb_ref[...], preferred_element_type=jnp.float32)
    out_ref[...] = x_ref[...] + agg + vel_ref[...] * scale


def _update(x128, p128, ic, vel128, vn, wv, bv, vv, c2b, bb_mat):
    return pl.pallas_call(
        _update_body,
        grid=(NL // UB,),
        in_specs=[
            pl.BlockSpec((UB, 128), lambda i: (i, 0)),
            pl.BlockSpec((NC, UB, 128), lambda i: (0, i, 0)),
            pl.BlockSpec((UB, 128), lambda i: (i, 0)),
            pl.BlockSpec((UB, 128), lambda i: (i, 0)),
            pl.BlockSpec((UB, 128), lambda i: (i, 0)),
            pl.BlockSpec((128, 1024), lambda i: (0, 0)),
            pl.BlockSpec((1, 1024), lambda i: (0, 0)),
            pl.BlockSpec((1024, 16), lambda i: (0, 0)),
            pl.BlockSpec((1, 1), lambda i: (0, 0)),
            pl.BlockSpec((16, 128), lambda i: (0, 0)),
        ],
        out_specs=pl.BlockSpec((UB, 128), lambda i: (i, 0)),
        out_shape=jax.ShapeDtypeStruct((NL, 128), jnp.float32),
    )(x128, p128, ic, vel128, vn, wv, bv, vv, c2b, bb_mat)


_I16 = np.eye(16, dtype=np.float32)


def _big_w(w1):
    blocks = []
    for t in range(8):
        p = jnp.pad(w1[8 * t:8 * t + 8, :].T, ((0, 3), (0, 0)))
        blocks.append(jnp.kron(_I16, p))
    return jnp.concatenate(blocks, axis=1)


def _big_b(b1):
    return jnp.concatenate(
        [jnp.tile(b1[8 * t:8 * t + 8], 16) for t in range(8)])[None, :]


def _big_v(v2):
    return jnp.concatenate(
        [jnp.kron(_I16, v2[8 * t:8 * t + 8][:, None]) for t in range(8)],
        axis=0)


def _ea_pack_body(ea_ref, m_ref, out_ref):
    planes = [jnp.reshape(ea_ref[j, :], (50, 128)) for j in range(4)]
    cat = jnp.concatenate(planes, axis=1)
    for t in range(8):
        out_ref[:, t, :] = jnp.dot(cat, m_ref[t],
                                   preferred_element_type=jnp.float32)


def _ea_pack_call(ea_t, m_mat):
    out = pl.pallas_call(
        _ea_pack_body,
        grid=(MLP_VALID,),
        in_specs=[
            pl.BlockSpec((4, E_BLK), lambda i: (0, i)),
            pl.BlockSpec((8, 512, 128), lambda i: (0, 0, 0)),
        ],
        out_specs=pl.BlockSpec((50, 8, 128), lambda i: (i, 0, 0)),
        out_shape=jax.ShapeDtypeStruct((E // 128, 8, 128), jnp.float32),
    )(ea_t, m_mat)
    return out.reshape(E // 16, 128)


def kernel(vel_norm, x, edges, vel, edge_attr, phi1_w, phi1_b, phi2_w,
           cmv1_w, cmv1_b, cmv2_w, cmv2_b):
    row = edges[0]
    col = edges[1]
    idx_g = jnp.concatenate(
        [row, col, jnp.zeros((G_PAD - 2 * E,), jnp.int32)])
    idx_s = jnp.concatenate([row, jnp.zeros((E_PAD - E,), jnp.int32)])
    zeros4 = jnp.zeros((N, DW), jnp.float32)
    zeros1 = jnp.zeros((N,), jnp.float32)

    s_mat = jnp.asarray(np.kron(_I16, np.ones((8, 1), np.float32)))
    r_mat = jnp.asarray(np.kron(_I16, np.eye(8, dtype=np.float32)[:1]))
    bb_mat = jnp.asarray(np.kron(_I16, np.ones((1, 8), np.float32)))

    x8 = jnp.pad(x, ((0, N_PAD - N), (0, DW - 3)))
    vel128 = jnp.pad(vel, ((0, N_PAD - N), (0, DW - 3))).reshape(NL, 128)
    vn128 = jnp.pad(vel_norm, ((0, N_PAD - N), (0, DW - 1))).reshape(NL, 128)
    m_np = np.zeros((8, 512, 128), np.float32)
    for t in range(8):
        for j in range(4):
            for k in range(16):
                m_np[t, 128 * j + 16 * t + k, 8 * k + 1 + j] = 1.0
    ea_pack = _ea_pack_call(edge_attr.T, jnp.asarray(m_np))

    cnt0, cnt1 = _count(idx_s, zeros1)
    cnt = cnt0 + cnt1
    cnt = cnt.at[0].add(-float(E_PAD - E))
    inv_cnt = jnp.pad(1.0 / jnp.clip(cnt, 1.0, None), (0, N_PAD - N))
    ic128 = jnp.repeat(inv_cnt.reshape(NL, 16), 8, axis=1)

    for i in range(4):
        gathered = _gather(x8, idx_g)
        g128 = gathered.reshape(GL, 128)
        w_big = _big_w(phi1_w[i])
        b_big = _big_b(phi1_b[i])
        v_big = _big_v(phi2_w[i][0])
        m128 = _edge_mlp(g128, ea_pack, w_big, b_big, v_big,
                         s_mat, r_mat, bb_mat)
        p = _scatter(m128.reshape(E_PAD, DW), idx_s, zeros4)
        wv = _big_w(jnp.pad(cmv1_w[i], ((0, 0), (0, 4))))
        bv = _big_b(cmv1_b[i])
        vv = _big_v(cmv2_w[i][0])
        x128 = _update(x8.reshape(NL, 128), p.reshape(NC, NL, 128),
                       ic128, vel128, vn128, wv, bv, vv,
                       cmv2_b[i][:, None], bb_mat)
        x8 = x128.reshape(N_PAD, DW)
    return x8[:N, :3]
